# baseline (speedup 1.0000x reference)
import functools

import jax
import jax.numpy as jnp
from jax.experimental import pallas as pl
from jax.experimental.pallas import tpu as pltpu

_VB = 32768


def _body(x_ref, lti_ref, am_ref, gt_ref, gi_ref,
          tok_out, lti_out, am_out, gt_out, gi_out,
          vmax_ref, vidx_ref, *, B, V, S, NB):
    i = pl.program_id(0)

    @pl.when(i == 0)
    def _init():
        vmax_ref[...] = jnp.full((B, 1), -jnp.inf, jnp.float32)
        vidx_ref[...] = jnp.zeros((B, 1), jnp.int32)

    def _update(x):
        bmax = jnp.max(x, axis=1, keepdims=True)
        lidx = jax.lax.broadcasted_iota(jnp.int32, (B, _VB), 1)
        cand = jnp.where(x == bmax, lidx, jnp.int32(_VB))
        bidx = jnp.min(cand, axis=1, keepdims=True) + i * _VB
        better = bmax > vmax_ref[...]
        vidx_ref[...] = jnp.where(better, bidx, vidx_ref[...])
        vmax_ref[...] = jnp.where(better, bmax, vmax_ref[...])

    @pl.when(i < NB - 1)
    def _full():
        _update(x_ref[...])

    @pl.when(i == NB - 1)
    def _tail():
        rem = V - (NB - 1) * _VB
        x = x_ref[...]
        lidx = jax.lax.broadcasted_iota(jnp.int32, (B, _VB), 1)
        x = jnp.where(lidx < rem, x, -jnp.inf)
        _update(x)

        tokens = vidx_ref[...]
        tok_out[...] = tokens
        lti = jnp.minimum(lti_ref[...] + 1, S - 1)
        lti_out[...] = lti
        scol = jax.lax.broadcasted_iota(jnp.int32, (B, S), 1)
        am_out[...] = jnp.where(scol == lti, 1, am_ref[...])
        gi = gi_ref[0]
        gt_out[...] = jnp.where(scol == gi, tokens, gt_ref[...])
        gi_out[0] = jnp.minimum(gi + 1, S - 1)


def kernel(logits, last_token_index, attention_mask, generated_tokens, generated_index):
    B, _, V = logits.shape
    S = generated_tokens.shape[1]
    NB = pl.cdiv(V, _VB)
    x2d = logits.reshape(B, V)

    grid = (NB,)
    out_shapes = (
        jax.ShapeDtypeStruct((B, 1), jnp.int32),
        jax.ShapeDtypeStruct((B, 1), jnp.int32),
        jax.ShapeDtypeStruct((B, S), attention_mask.dtype),
        jax.ShapeDtypeStruct((B, S), generated_tokens.dtype),
        jax.ShapeDtypeStruct((1,), jnp.int32),
    )
    const = lambda i: (0, 0)
    in_specs = [
        pl.BlockSpec((B, _VB), lambda i: (0, i)),
        pl.BlockSpec((B, 1), const),
        pl.BlockSpec((B, S), const),
        pl.BlockSpec((B, S), const),
        pl.BlockSpec(memory_space=pltpu.SMEM),
    ]
    out_specs = [
        pl.BlockSpec((B, 1), const),
        pl.BlockSpec((B, 1), const),
        pl.BlockSpec((B, S), const),
        pl.BlockSpec((B, S), const),
        pl.BlockSpec(memory_space=pltpu.SMEM),
    ]
    tok, lti, am, gt, gi = pl.pallas_call(
        functools.partial(_body, B=B, V=V, S=S, NB=NB),
        grid=grid,
        in_specs=in_specs,
        out_specs=out_specs,
        out_shape=out_shapes,
        scratch_shapes=[
            pltpu.VMEM((B, 1), jnp.float32),
            pltpu.VMEM((B, 1), jnp.int32),
        ],
        compiler_params=pltpu.CompilerParams(
            dimension_semantics=("arbitrary",),
        ),
    )(x2d, last_token_index, attention_mask, generated_tokens, generated_index)
    return tok, lti, am, gt, gi
